# Initial kernel scaffold; baseline (speedup 1.0000x reference)
import functools

import jax
import jax.numpy as jnp
from jax import lax
from jax.experimental import pallas as pl
from jax.experimental.pallas import tpu as pltpu
from jax.experimental.pallas import tpu_sc as plsc

N = 10000
E = 320000
D = 128
H1 = 256
H2 = 64

NS = 16
NC = 2
EPT = E // NS
K = 80
C = EPT // K
RPS = N // NS
L = 16


def _spmm_sc(table0, table1, src3, dst3, w3, zrows, F):
    mesh = plsc.VectorSubcoreMesh(core_axis_name="c", subcore_axis_name="s")

    def body(t0, t1, src_h, dst_h, w_h, z_h, out0, out1,
             src_v, dst_v, w_v, rows_v, acc, gsem):
        c = lax.axis_index("c")
        s = lax.axis_index("s")

        pltpu.sync_copy(z_h, acc.at[pl.ds(s * RPS, RPS)])
        pltpu.sync_copy(src_h.at[s], src_v)
        pltpu.sync_copy(dst_h.at[s], dst_v)
        pltpu.sync_copy(w_h.at[s], w_v)
        plsc.subcore_barrier()

        def edge_loop(tbl):
            def chunk(j, carry):
                pltpu.async_copy(tbl.at[src_v.at[j]], rows_v, gsem).wait()

                def group(g, carry2):
                    w16 = w_v[j, pl.ds(g * L, L)]
                    for l in range(L):
                        wb = jnp.take(w16, jnp.full((L,), l, jnp.int32),
                                      mode="promise_in_bounds")
                        for fv in range(F // L):
                            sl = pl.ds(fv * L, L)
                            rows_v[g * L + l, sl] = rows_v[g * L + l, sl] * wb
                    return carry2

                lax.fori_loop(0, K // L, group, 0, unroll=False)

                pltpu.sync_copy(rows_v, acc.at[dst_v.at[j]], add=True)
                return carry

            lax.fori_loop(0, C, chunk, 0, unroll=False)

        @pl.when(c == 0)
        def _():
            edge_loop(t0)

        @pl.when(c == 1)
        def _():
            edge_loop(t1)

        plsc.subcore_barrier()

        sl = pl.ds(s * RPS, RPS)

        @pl.when(c == 0)
        def _():
            pltpu.sync_copy(acc.at[sl], out0.at[sl])

        @pl.when(c == 1)
        def _():
            pltpu.sync_copy(acc.at[sl], out1.at[sl])

    kern = pl.kernel(
        body,
        out_type=(jax.ShapeDtypeStruct((N, F), jnp.float32),
                  jax.ShapeDtypeStruct((N, F), jnp.float32)),
        mesh=mesh,
        scratch_types=[
            pltpu.VMEM((C, K), jnp.int32),
            pltpu.VMEM((C, K), jnp.int32),
            pltpu.VMEM((C, K), jnp.float32),
            pltpu.VMEM((K, F), jnp.float32),
            pltpu.VMEM_SHARED((N, F), jnp.float32),
            pltpu.SemaphoreType.DMA,
        ],
    )
    return kern(table0, table1, src3, dst3, w3, zrows)


def _proj1_body(x_ref, w_ref, oa_ref, ob_ref):
    h = jnp.dot(x_ref[...], w_ref[...], preferred_element_type=jnp.float32)
    oa_ref[...] = h[:, :D]
    ob_ref[...] = h[:, D:]


def _proj1(x, W0):
    grid = 10
    bm = N // grid
    return pl.pallas_call(
        _proj1_body,
        grid=(grid,),
        in_specs=[
            pl.BlockSpec((bm, D), lambda i: (i, 0)),
            pl.BlockSpec((D, H1), lambda i: (0, 0)),
        ],
        out_specs=(
            pl.BlockSpec((bm, D), lambda i: (i, 0)),
            pl.BlockSpec((bm, D), lambda i: (i, 0)),
        ),
        out_shape=(jax.ShapeDtypeStruct((N, D), jnp.float32),
                   jax.ShapeDtypeStruct((N, D), jnp.float32)),
    )(x, W0)


def _proj2_body(sa_ref, sb_ref, w_ref, oa_ref, ob_ref):
    h = jnp.concatenate(
        [jnp.maximum(sa_ref[...], 0.0), jnp.maximum(sb_ref[...], 0.0)], axis=1)
    p = jnp.dot(h, w_ref[...], preferred_element_type=jnp.float32)
    oa_ref[...] = p[:, :H2]
    ob_ref[...] = p[:, H2:]


def _proj2(s1a, s1b, Wcat):
    grid = 10
    bm = N // grid
    return pl.pallas_call(
        _proj2_body,
        grid=(grid,),
        in_specs=[
            pl.BlockSpec((bm, D), lambda i: (i, 0)),
            pl.BlockSpec((bm, D), lambda i: (i, 0)),
            pl.BlockSpec((H1, 2 * H2), lambda i: (0, 0)),
        ],
        out_specs=(
            pl.BlockSpec((bm, H2), lambda i: (i, 0)),
            pl.BlockSpec((bm, H2), lambda i: (i, 0)),
        ),
        out_shape=(jax.ShapeDtypeStruct((N, H2), jnp.float32),
                   jax.ShapeDtypeStruct((N, H2), jnp.float32)),
    )(s1a, s1b, Wcat)


def _z_body(zm_ref, zl_ref, eps_ref, z_ref):
    z_ref[...] = zm_ref[...] + eps_ref[...] * jnp.exp(zl_ref[...])


def _z_compute(z_mean, z_log, eps):
    grid = 10
    bm = N // grid
    return pl.pallas_call(
        _z_body,
        grid=(grid,),
        in_specs=[pl.BlockSpec((bm, H2), lambda i: (i, 0))] * 3,
        out_specs=pl.BlockSpec((bm, H2), lambda i: (i, 0)),
        out_shape=jax.ShapeDtypeStruct((N, H2), jnp.float32),
    )(z_mean, z_log, eps)


def _dec_body(zi_ref, zmi_ref, zj_ref, zmj_ref, r_ref, rn_ref):
    dims = (((1,), (1,)), ((), ()))
    r_ref[...] = lax.dot_general(zi_ref[...], zj_ref[...], dims,
                                 preferred_element_type=jnp.float32)
    rn_ref[...] = lax.dot_general(zmi_ref[...], zmj_ref[...], dims,
                                  preferred_element_type=jnp.float32)


def _decoder(z, z_mean):
    bm = 1024
    grid = pl.cdiv(N, bm)
    return pl.pallas_call(
        _dec_body,
        grid=(grid, grid),
        in_specs=[
            pl.BlockSpec((bm, H2), lambda i, j: (i, 0)),
            pl.BlockSpec((bm, H2), lambda i, j: (i, 0)),
            pl.BlockSpec((bm, H2), lambda i, j: (j, 0)),
            pl.BlockSpec((bm, H2), lambda i, j: (j, 0)),
        ],
        out_specs=(
            pl.BlockSpec((bm, bm), lambda i, j: (i, j)),
            pl.BlockSpec((bm, bm), lambda i, j: (i, j)),
        ),
        out_shape=(jax.ShapeDtypeStruct((N, N), jnp.float32),
                   jax.ShapeDtypeStruct((N, N), jnp.float32)),
    )(z, z_mean, z, z_mean)


def kernel(x, edge_index, edge_weight, eps, W0, W_mu, W_logstd):
    src3 = edge_index[0].reshape(NS, C, K)
    dst3 = edge_index[1].reshape(NS, C, K)
    w3 = edge_weight.reshape(NS, C, K)
    Wcat = jnp.concatenate([W_mu, W_logstd], axis=1)
    zrows_d = jnp.zeros((RPS, D), jnp.float32)
    zrows_h = jnp.zeros((RPS, H2), jnp.float32)

    h0a, h0b = _proj1(x, W0)
    s1a, s1b = _spmm_sc(h0a, h0b, src3, dst3, w3, zrows_d, D)
    pa, pb = _proj2(s1a, s1b, Wcat)
    z_mean, z_log = _spmm_sc(pa, pb, src3, dst3, w3, zrows_h, H2)
    z = _z_compute(z_mean, z_log, eps)
    recon, recon_nl = _decoder(z, z_mean)
    return recon.reshape(-1), recon_nl.reshape(-1)


# speedup vs baseline: 2.5025x; 2.5025x over previous
import jax
import jax.numpy as jnp
from jax import lax
from jax.experimental import pallas as pl
from jax.experimental.pallas import tpu as pltpu
from jax.experimental.pallas import tpu_sc as plsc

N = 10000
E = 320000
D = 128
H1 = 256
H2 = 64

NS = 16
NC = 2
K = 128
L = 16
RPS = 624
TAIL = N - NS * RPS

def _round_up(v, m):
    return -(-v // m) * m


C1 = _round_up(-(-(E // NS) // K), 8)
C2 = _round_up(-(-(E // (NS * NC)) // K), 8)
SC1 = C1 // 8
SC2 = C2 // 8

_GATHER_DN = lax.GatherDimensionNumbers(
    offset_dims=(), collapsed_slice_dims=(0,), start_index_map=(0,))


def _bcast_lane(v, l):
    idx = jnp.full((L, 1), l, jnp.int32)
    return lax.gather(v, idx, _GATHER_DN, (1,),
                      mode=lax.GatherScatterMode.PROMISE_IN_BOUNDS)


def _spmm_sc(table0, table1, src4, dst4, w4, zrows, nsuper, split_edges):
    mesh = plsc.VectorSubcoreMesh(core_axis_name="c", subcore_axis_name="s")

    def body(t0, t1, src_h, dst_h, w_h, z_h, out0, out1,
             src_v, dst_v, w_v, rows_v, acc, gsem):
        c = lax.axis_index("c")
        s = lax.axis_index("s")

        pltpu.sync_copy(z_h.at[pl.ds(0, RPS)], acc.at[pl.ds(s * RPS, RPS)])

        @pl.when(s == NS - 1)
        def _():
            pltpu.sync_copy(z_h.at[pl.ds(RPS, TAIL)],
                            acc.at[pl.ds(NS * RPS, TAIL)])

        erow = s * NC + c if split_edges else s
        plsc.subcore_barrier()

        def edge_loop(tbl):
            def superchunk(u, carry):
                pltpu.sync_copy(src_h.at[erow, u], src_v)
                pltpu.sync_copy(dst_h.at[erow, u], dst_v)
                pltpu.sync_copy(w_h.at[erow, u], w_v)

                for j in range(8):
                    pltpu.async_copy(tbl.at[src_v.at[j]], rows_v, gsem).wait()

                    def group(g, carry2):
                        w16 = w_v[j, pl.ds(g * L, L)]
                        for l in range(L):
                            wb = _bcast_lane(w16, l)
                            for fv in range(D // L):
                                sl = pl.ds(fv * L, L)
                                rows_v[g * L + l, sl] = (
                                    rows_v[g * L + l, sl] * wb)
                        return carry2

                    lax.fori_loop(0, K // L, group, 0, unroll=False)

                    pltpu.sync_copy(rows_v, acc.at[dst_v.at[j]], add=True)
                return carry

            lax.fori_loop(0, nsuper, superchunk, 0, unroll=False)

        @pl.when(c == 0)
        def _():
            edge_loop(t0)

        @pl.when(c == 1)
        def _():
            edge_loop(t1)

        plsc.subcore_barrier()

        sl = pl.ds(s * RPS, RPS)
        tl = pl.ds(NS * RPS, TAIL)
        last = s == NS - 1

        @pl.when(c == 0)
        def _():
            pltpu.sync_copy(acc.at[sl], out0.at[sl])

            @pl.when(last)
            def _():
                pltpu.sync_copy(acc.at[tl], out0.at[tl])

        @pl.when(c == 1)
        def _():
            pltpu.sync_copy(acc.at[sl], out1.at[sl])

            @pl.when(last)
            def _():
                pltpu.sync_copy(acc.at[tl], out1.at[tl])

    kern = pl.kernel(
        body,
        out_type=(jax.ShapeDtypeStruct((N, D), jnp.float32),
                  jax.ShapeDtypeStruct((N, D), jnp.float32)),
        mesh=mesh,
        scratch_types=[
            pltpu.VMEM((8, K), jnp.int32),
            pltpu.VMEM((8, K), jnp.int32),
            pltpu.VMEM((8, K), jnp.float32),
            pltpu.VMEM((K, D), jnp.float32),
            pltpu.VMEM_SHARED((N, D), jnp.float32),
            pltpu.SemaphoreType.DMA,
        ],
    )
    return kern(table0, table1, src4, dst4, w4, zrows)


def _pad_edges(src, dst, w, rows, nsuper):
    tot = rows * nsuper * 8 * K
    pad = tot - src.shape[0]
    src = jnp.concatenate([src, jnp.zeros((pad,), src.dtype)])
    dst = jnp.concatenate([dst, jnp.zeros((pad,), dst.dtype)])
    w = jnp.concatenate([w, jnp.zeros((pad,), w.dtype)])
    return (src.reshape(rows, nsuper, 8, K), dst.reshape(rows, nsuper, 8, K),
            w.reshape(rows, nsuper, 8, K))


def _proj1_body(x_ref, w_ref, oa_ref, ob_ref):
    h = jnp.dot(x_ref[...], w_ref[...], preferred_element_type=jnp.float32)
    oa_ref[...] = h[:, :D]
    ob_ref[...] = h[:, D:]


def _proj1(x, W0):
    grid = 10
    bm = N // grid
    return pl.pallas_call(
        _proj1_body,
        grid=(grid,),
        in_specs=[
            pl.BlockSpec((bm, D), lambda i: (i, 0)),
            pl.BlockSpec((D, H1), lambda i: (0, 0)),
        ],
        out_specs=(
            pl.BlockSpec((bm, D), lambda i: (i, 0)),
            pl.BlockSpec((bm, D), lambda i: (i, 0)),
        ),
        out_shape=(jax.ShapeDtypeStruct((N, D), jnp.float32),
                   jax.ShapeDtypeStruct((N, D), jnp.float32)),
    )(x, W0)


def _proj2_body(sa_ref, sb_ref, w_ref, o_ref):
    h = jnp.concatenate(
        [jnp.maximum(sa_ref[...], 0.0), jnp.maximum(sb_ref[...], 0.0)], axis=1)
    o_ref[...] = jnp.dot(h, w_ref[...], preferred_element_type=jnp.float32)


def _proj2(s1a, s1b, Wcat):
    grid = 10
    bm = N // grid
    return pl.pallas_call(
        _proj2_body,
        grid=(grid,),
        in_specs=[
            pl.BlockSpec((bm, D), lambda i: (i, 0)),
            pl.BlockSpec((bm, D), lambda i: (i, 0)),
            pl.BlockSpec((H1, 2 * H2), lambda i: (0, 0)),
        ],
        out_specs=pl.BlockSpec((bm, 2 * H2), lambda i: (i, 0)),
        out_shape=jax.ShapeDtypeStruct((N, 2 * H2), jnp.float32),
    )(s1a, s1b, Wcat)


def _z_body(p0_ref, p1_ref, eps_ref, z_ref, zm_ref):
    p = p0_ref[...] + p1_ref[...]
    zm = p[:, :H2]
    zl = p[:, H2:]
    zm_ref[...] = zm
    z_ref[...] = zm + eps_ref[...] * jnp.exp(zl)


def _z_compute(part0, part1, eps):
    grid = 10
    bm = N // grid
    return pl.pallas_call(
        _z_body,
        grid=(grid,),
        in_specs=[
            pl.BlockSpec((bm, 2 * H2), lambda i: (i, 0)),
            pl.BlockSpec((bm, 2 * H2), lambda i: (i, 0)),
            pl.BlockSpec((bm, H2), lambda i: (i, 0)),
        ],
        out_specs=(
            pl.BlockSpec((bm, H2), lambda i: (i, 0)),
            pl.BlockSpec((bm, H2), lambda i: (i, 0)),
        ),
        out_shape=(jax.ShapeDtypeStruct((N, H2), jnp.float32),
                   jax.ShapeDtypeStruct((N, H2), jnp.float32)),
    )(part0, part1, eps)


def _dec_body(zi_ref, zmi_ref, zj_ref, zmj_ref, r_ref, rn_ref):
    dims = (((1,), (1,)), ((), ()))
    r_ref[...] = lax.dot_general(zi_ref[...], zj_ref[...], dims,
                                 preferred_element_type=jnp.float32)
    rn_ref[...] = lax.dot_general(zmi_ref[...], zmj_ref[...], dims,
                                  preferred_element_type=jnp.float32)


def _decoder(z, z_mean):
    bm = 1024
    grid = pl.cdiv(N, bm)
    return pl.pallas_call(
        _dec_body,
        grid=(grid, grid),
        in_specs=[
            pl.BlockSpec((bm, H2), lambda i, j: (i, 0)),
            pl.BlockSpec((bm, H2), lambda i, j: (i, 0)),
            pl.BlockSpec((bm, H2), lambda i, j: (j, 0)),
            pl.BlockSpec((bm, H2), lambda i, j: (j, 0)),
        ],
        out_specs=(
            pl.BlockSpec((bm, bm), lambda i, j: (i, j)),
            pl.BlockSpec((bm, bm), lambda i, j: (i, j)),
        ),
        out_shape=(jax.ShapeDtypeStruct((N, N), jnp.float32),
                   jax.ShapeDtypeStruct((N, N), jnp.float32)),
    )(z, z_mean, z, z_mean)


def kernel(x, edge_index, edge_weight, eps, W0, W_mu, W_logstd):
    src = edge_index[0]
    dst = edge_index[1]
    s1_src, s1_dst, s1_w = _pad_edges(src, dst, edge_weight, NS, SC1)
    s2_src, s2_dst, s2_w = _pad_edges(src, dst, edge_weight, NS * NC, SC2)
    Wcat = jnp.concatenate([W_mu, W_logstd], axis=1)
    zrows = jnp.zeros((RPS + TAIL, D), jnp.float32)

    h0a, h0b = _proj1(x, W0)
    s1a, s1b = _spmm_sc(h0a, h0b, s1_src, s1_dst, s1_w, zrows, SC1, False)
    p = _proj2(s1a, s1b, Wcat)
    q0, q1 = _spmm_sc(p, p, s2_src, s2_dst, s2_w, zrows, SC2, True)
    z, z_mean = _z_compute(q0, q1, eps)
    recon, recon_nl = _decoder(z, z_mean)
    return recon.reshape(-1), recon_nl.reshape(-1)
